# baseline (speedup 1.0000x reference)
import functools

import jax
import jax.numpy as jnp
import numpy as np
from jax.experimental import pallas as pl
from jax.experimental.pallas import tpu as pltpu
from jax.experimental.pallas import tpu_sc as plsc

N = 10000
D = 128
E = 320000

NC = 2
NS = 16
NW = NC * NS
L = 16

CHUNK = 64
NBUF = 4
OFF = NBUF // 2
NSTAGE = 4
NCHT = NSTAGE * NBUF * NW * -(-E // (CHUNK * NSTAGE * NBUF * NW))
CPT = NCHT // NW
SCH = CPT // NSTAGE
EPAD = NCHT * CHUNK
NPAD = -(-N // (NS * 128)) * (NS * 128)
NPT = NPAD // NS
DEG_G = 8

_k = np.arange(EPAD - E, dtype=np.int32)
_PAD_EDGES = np.stack([_k % N, N + _k % (NPAD - N)]).astype(np.int32)

_mesh = functools.partial(
    plsc.VectorSubcoreMesh,
    core_axis_name="c",
    subcore_axis_name="s",
    num_cores=NC,
    num_subcores=NS,
)


def _deg_body(eic, deg_out, cidx_all, onesbuf, zbuf, sh_deg, sem):
    cid = jax.lax.axis_index("c")
    sid = jax.lax.axis_index("s")
    wid = sid * NC + cid

    @pl.loop(0, CHUNK // L)
    def _(i):
        onesbuf[pl.ds(i * L, L)] = jnp.ones((L,), jnp.float32)

    @pl.loop(0, NPT // L)
    def _(i):
        zbuf[pl.ds(i * L, L)] = jnp.zeros((L,), jnp.float32)

    pltpu.sync_copy(zbuf, sh_deg.at[pl.ds(sid * NPT, NPT)])
    plsc.subcore_barrier()

    for h in range(NSTAGE):
        pltpu.sync_copy(eic.at[1, pl.ds(wid * CPT + h * SCH, SCH)], cidx_all)

        @pl.loop(0, SCH, step=DEG_G)
        def _(g):
            for k in range(DEG_G):
                pltpu.async_copy(onesbuf, sh_deg.at[cidx_all.at[g + k]], sem, add=True)
            for k in range(DEG_G):
                pltpu.make_async_copy(onesbuf, sh_deg.at[cidx_all.at[g + k]], sem).wait()

    plsc.subcore_barrier()
    pltpu.sync_copy(
        sh_deg.at[pl.ds(sid * NPT, NPT)], deg_out.at[cid, pl.ds(sid * NPT, NPT)]
    )


def _deg_partials(eic):
    return pl.kernel(
        _deg_body,
        out_type=jax.ShapeDtypeStruct((NC, NPAD), jnp.float32),
        mesh=_mesh(),
        scratch_types=[
            pltpu.VMEM((SCH, CHUNK), jnp.int32),
            pltpu.VMEM((CHUNK,), jnp.float32),
            pltpu.VMEM((NPT,), jnp.float32),
            pltpu.VMEM_SHARED((NPAD,), jnp.float32),
            pltpu.SemaphoreType.DMA,
        ],
    )(eic)


def _scale_body(dp_ref, emb_ref, dis_ref, tmp_ref):
    deg = dp_ref[0] + dp_ref[1]
    dis = jnp.where(deg > 0.0, jax.lax.rsqrt(deg), 0.0)
    dis_ref[...] = dis
    tmp_ref[...] = dis[:N] * emb_ref[...]


def _scale(deg_partials, emb):
    return pl.pallas_call(
        _scale_body,
        out_shape=[
            jax.ShapeDtypeStruct((NPAD, 1), jnp.float32),
            jax.ShapeDtypeStruct((N, D), jnp.float32),
        ],
    )(deg_partials.reshape(NC, NPAD, 1), emb)


def _prop_body(tmp, eic, acc_out, ridx_st, cidx_st, rbufs, gsems, ssems, sh_acc):
    cid = jax.lax.axis_index("c")
    sid = jax.lax.axis_index("s")
    wid = sid * NC + cid

    @pl.loop(0, CHUNK)
    def _(r):
        for c in range(D // L):
            rbufs[0][r, pl.ds(c * L, L)] = jnp.zeros((L,), jnp.float32)

    for k in range(NPT // CHUNK):
        pltpu.sync_copy(rbufs[0], sh_acc.at[pl.ds(sid * NPT + k * CHUNK, CHUNK)])
    plsc.subcore_barrier()

    def gather(i, b):
        pltpu.async_copy(tmp.at[ridx_st.at[i]], rbufs[b], gsems[b])

    def gather_wait(i, b):
        pltpu.make_async_copy(tmp.at[ridx_st.at[i]], rbufs[b], gsems[b]).wait()

    def scatter(i, b):
        pltpu.async_copy(rbufs[b], sh_acc.at[cidx_st.at[i]], ssems[b], add=True)

    def scatter_wait(i, b):
        pltpu.make_async_copy(rbufs[b], sh_acc.at[cidx_st.at[i]], ssems[b]).wait()

    for h in range(NSTAGE):
        off = wid * CPT + h * SCH
        pltpu.sync_copy(eic.at[0, pl.ds(off, SCH)], ridx_st)
        pltpu.sync_copy(eic.at[1, pl.ds(off, SCH)], cidx_st)

        for b in range(OFF):
            gather(b, b)

        @pl.loop(0, SCH, step=NBUF)
        def _(g):
            for b in range(NBUF):
                i = g + b
                b2 = (b + OFF) % NBUF

                @pl.when(i >= OFF)
                def _():
                    scatter_wait(i - OFF, b2)

                @pl.when(i + OFF < SCH)
                def _():
                    gather(i + OFF, b2)

                gather_wait(i, b)
                scatter(i, b)

        for j in range(SCH - OFF, SCH):
            scatter_wait(j, j % NBUF)

    plsc.subcore_barrier()
    for k in range(NPT // CHUNK):
        r0 = sid * NPT + k * CHUNK
        pltpu.sync_copy(sh_acc.at[pl.ds(r0, CHUNK)], acc_out.at[cid, pl.ds(r0, CHUNK)])


def _propagate(tmp, eic):
    return pl.kernel(
        _prop_body,
        out_type=jax.ShapeDtypeStruct((NC, NPAD, D), jnp.float32),
        mesh=_mesh(),
        scratch_types=[
            pltpu.VMEM((SCH, CHUNK), jnp.int32),
            pltpu.VMEM((SCH, CHUNK), jnp.int32),
            [pltpu.VMEM((CHUNK, D), jnp.float32) for _ in range(NBUF)],
            [pltpu.SemaphoreType.DMA for _ in range(NBUF)],
            [pltpu.SemaphoreType.DMA for _ in range(NBUF)],
            pltpu.VMEM_SHARED((NPAD, D), jnp.float32),
        ],
    )(tmp, eic)


def _combine_body(ap_ref, dis_ref, out_ref):
    s = ap_ref[0, :N, :] + ap_ref[1, :N, :]
    out_ref[...] = dis_ref[:N] * s


def _combine(acc_partials, dis):
    return pl.pallas_call(
        _combine_body,
        out_shape=jax.ShapeDtypeStruct((N, D), jnp.float32),
    )(acc_partials, dis)


@jax.jit
def kernel(emb, edge_index):
    ei = edge_index.astype(jnp.int32)
    eic = jnp.concatenate([ei, jnp.asarray(_PAD_EDGES)], axis=1)
    eic = eic.reshape(2, NCHT, CHUNK)
    dp = _deg_partials(eic)
    dis, tmp = _scale(dp, emb)
    acc = _propagate(tmp, eic)
    return _combine(acc, dis)


# speedup vs baseline: 1.0664x; 1.0664x over previous
import functools

import jax
import jax.numpy as jnp
import numpy as np
from jax.experimental import pallas as pl
from jax.experimental.pallas import tpu as pltpu
from jax.experimental.pallas import tpu_sc as plsc

N = 10000
D = 128
E = 320000

NC = 2
NS = 16
NW = NC * NS
L = 16

CHUNK = 128
NBUF = 2
OFF = NBUF // 2
NSTAGE = 2
NCHT = NSTAGE * NBUF * NW * -(-E // (CHUNK * NSTAGE * NBUF * NW))
CPT = NCHT // NW
SCH = CPT // NSTAGE
EPAD = NCHT * CHUNK
NPAD = -(-N // (NS * 128)) * (NS * 128)
NPT = NPAD // NS
DEG_G = 8

_k = np.arange(EPAD - E, dtype=np.int32)
_PAD_EDGES = np.stack([_k % N, N + _k % (NPAD - N)]).astype(np.int32)

_mesh = functools.partial(
    plsc.VectorSubcoreMesh,
    core_axis_name="c",
    subcore_axis_name="s",
    num_cores=NC,
    num_subcores=NS,
)


def _deg_body(eic, deg_out, cidx_all, onesbuf, zbuf, sh_deg, sem):
    cid = jax.lax.axis_index("c")
    sid = jax.lax.axis_index("s")
    wid = sid * NC + cid

    @pl.loop(0, CHUNK // L)
    def _(i):
        onesbuf[pl.ds(i * L, L)] = jnp.ones((L,), jnp.float32)

    @pl.loop(0, NPT // L)
    def _(i):
        zbuf[pl.ds(i * L, L)] = jnp.zeros((L,), jnp.float32)

    pltpu.sync_copy(zbuf, sh_deg.at[pl.ds(sid * NPT, NPT)])
    plsc.subcore_barrier()

    for h in range(NSTAGE):
        pltpu.sync_copy(eic.at[1, pl.ds(wid * CPT + h * SCH, SCH)], cidx_all)

        @pl.loop(0, SCH, step=DEG_G)
        def _(g):
            for k in range(DEG_G):
                pltpu.async_copy(onesbuf, sh_deg.at[cidx_all.at[g + k]], sem, add=True)
            for k in range(DEG_G):
                pltpu.make_async_copy(onesbuf, sh_deg.at[cidx_all.at[g + k]], sem).wait()

    plsc.subcore_barrier()
    pltpu.sync_copy(
        sh_deg.at[pl.ds(sid * NPT, NPT)], deg_out.at[cid, pl.ds(sid * NPT, NPT)]
    )


def _deg_partials(eic):
    return pl.kernel(
        _deg_body,
        out_type=jax.ShapeDtypeStruct((NC, NPAD), jnp.float32),
        mesh=_mesh(),
        scratch_types=[
            pltpu.VMEM((SCH, CHUNK), jnp.int32),
            pltpu.VMEM((CHUNK,), jnp.float32),
            pltpu.VMEM((NPT,), jnp.float32),
            pltpu.VMEM_SHARED((NPAD,), jnp.float32),
            pltpu.SemaphoreType.DMA,
        ],
    )(eic)


def _scale_body(dp_ref, emb_ref, dis_ref, tmp_ref):
    deg = dp_ref[0] + dp_ref[1]
    dis = jnp.where(deg > 0.0, jax.lax.rsqrt(deg), 0.0)
    dis_ref[...] = dis
    tmp_ref[...] = dis[:N] * emb_ref[...]


def _scale(deg_partials, emb):
    return pl.pallas_call(
        _scale_body,
        out_shape=[
            jax.ShapeDtypeStruct((NPAD, 1), jnp.float32),
            jax.ShapeDtypeStruct((N, D), jnp.float32),
        ],
    )(deg_partials.reshape(NC, NPAD, 1), emb)


def _prop_body(tmp, eic, acc_out, ridx_st, cidx_st, rbufs, gsems, ssems, sh_acc):
    cid = jax.lax.axis_index("c")
    sid = jax.lax.axis_index("s")
    wid = sid * NC + cid

    @pl.loop(0, CHUNK)
    def _(r):
        for c in range(D // L):
            rbufs[0][r, pl.ds(c * L, L)] = jnp.zeros((L,), jnp.float32)

    for k in range(NPT // CHUNK):
        pltpu.sync_copy(rbufs[0], sh_acc.at[pl.ds(sid * NPT + k * CHUNK, CHUNK)])
    plsc.subcore_barrier()

    def gather(i, b):
        pltpu.async_copy(tmp.at[ridx_st.at[i]], rbufs[b], gsems[b])

    def gather_wait(i, b):
        pltpu.make_async_copy(tmp.at[ridx_st.at[i]], rbufs[b], gsems[b]).wait()

    def scatter(i, b):
        pltpu.async_copy(rbufs[b], sh_acc.at[cidx_st.at[i]], ssems[b], add=True)

    def scatter_wait(i, b):
        pltpu.make_async_copy(rbufs[b], sh_acc.at[cidx_st.at[i]], ssems[b]).wait()

    for h in range(NSTAGE):
        off = wid * CPT + h * SCH
        pltpu.sync_copy(eic.at[0, pl.ds(off, SCH)], ridx_st)
        pltpu.sync_copy(eic.at[1, pl.ds(off, SCH)], cidx_st)

        for b in range(OFF):
            gather(b, b)

        @pl.loop(0, SCH, step=NBUF)
        def _(g):
            for b in range(NBUF):
                i = g + b
                b2 = (b + OFF) % NBUF

                @pl.when(i >= OFF)
                def _():
                    scatter_wait(i - OFF, b2)

                @pl.when(i + OFF < SCH)
                def _():
                    gather(i + OFF, b2)

                gather_wait(i, b)
                scatter(i, b)

        for j in range(SCH - OFF, SCH):
            scatter_wait(j, j % NBUF)

    plsc.subcore_barrier()
    for k in range(NPT // CHUNK):
        r0 = sid * NPT + k * CHUNK
        pltpu.sync_copy(sh_acc.at[pl.ds(r0, CHUNK)], acc_out.at[cid, pl.ds(r0, CHUNK)])


def _propagate(tmp, eic):
    return pl.kernel(
        _prop_body,
        out_type=jax.ShapeDtypeStruct((NC, NPAD, D), jnp.float32),
        mesh=_mesh(),
        scratch_types=[
            pltpu.VMEM((SCH, CHUNK), jnp.int32),
            pltpu.VMEM((SCH, CHUNK), jnp.int32),
            [pltpu.VMEM((CHUNK, D), jnp.float32) for _ in range(NBUF)],
            [pltpu.SemaphoreType.DMA for _ in range(NBUF)],
            [pltpu.SemaphoreType.DMA for _ in range(NBUF)],
            pltpu.VMEM_SHARED((NPAD, D), jnp.float32),
        ],
    )(tmp, eic)


def _combine_body(ap_ref, dis_ref, out_ref):
    s = ap_ref[0, :N, :] + ap_ref[1, :N, :]
    out_ref[...] = dis_ref[:N] * s


def _combine(acc_partials, dis):
    return pl.pallas_call(
        _combine_body,
        out_shape=jax.ShapeDtypeStruct((N, D), jnp.float32),
    )(acc_partials, dis)


@jax.jit
def kernel(emb, edge_index):
    ei = edge_index.astype(jnp.int32)
    eic = jnp.concatenate([ei, jnp.asarray(_PAD_EDGES)], axis=1)
    eic = eic.reshape(2, NCHT, CHUNK)
    dp = _deg_partials(eic)
    dis, tmp = _scale(dp, emb)
    acc = _propagate(tmp, eic)
    return _combine(acc, dis)
